# baseline (speedup 1.0000x reference)
import functools

import jax
import jax.numpy as jnp
from jax import lax
from jax.experimental import pallas as pl
from jax.experimental.pallas import tpu as pltpu
from jax.experimental.pallas import tpu_sc as plsc

D = 100
DP = 128
NC = 2
NS = 16
NW = NC * NS
G = 128
K = 2
CHUNK = K * G


def _make_gather(n_rows):
    rows_per_w = n_rows // NW
    ng = rows_per_w // G
    nchunk = ng // K
    assert n_rows % (NW * G) == 0 and ng % (2 * K) == 0

    mesh = plsc.VectorSubcoreMesh(core_axis_name="c", subcore_axis_name="s")

    @functools.partial(
        pl.kernel,
        out_type=jax.ShapeDtypeStruct((n_rows, DP), jnp.float32),
        mesh=mesh,
        scratch_types=[
            pltpu.VMEM((ng, G), jnp.int32),
            pltpu.VMEM((CHUNK, DP), jnp.float32),
            pltpu.VMEM((CHUNK, DP), jnp.float32),
            pltpu.SemaphoreType.DMA,
            pltpu.SemaphoreType.DMA,
            pltpu.SemaphoreType.DMA,
            pltpu.SemaphoreType.DMA,
        ],
    )
    def gather_kernel(idx_hbm, table_hbm, out_hbm,
                      idx_v, buf0, buf1, gsem0, gsem1, wsem0, wsem1):
        wid = lax.axis_index("s") * NC + lax.axis_index("c")
        row0 = wid * rows_per_w

        pltpu.sync_copy(idx_hbm.at[wid], idx_v)

        def out_slice(c):
            return out_hbm.at[pl.ds(row0 + c * CHUNK, CHUNK)]

        def fire_gathers(c, buf, sem):
            return [
                pltpu.async_copy(
                    table_hbm.at[idx_v.at[c * K + j]],
                    buf.at[pl.ds(j * G, G)],
                    sem,
                )
                for j in range(K)
            ]

        def body(t, _):
            a = 2 * t

            @pl.when(t > 0)
            def _drain_prev():
                pltpu.make_async_copy(buf0, out_slice(a - 2), wsem0).wait()
                pltpu.make_async_copy(buf1, out_slice(a - 1), wsem1).wait()

            ha = fire_gathers(a, buf0, gsem0)
            hb = fire_gathers(a + 1, buf1, gsem1)
            for h in ha:
                h.wait()
            pltpu.async_copy(buf0, out_slice(a), wsem0)
            for h in hb:
                h.wait()
            pltpu.async_copy(buf1, out_slice(a + 1), wsem1)
            return 0

        lax.fori_loop(0, nchunk // 2, body, 0)
        pltpu.make_async_copy(buf0, out_slice(nchunk - 2), wsem0).wait()
        pltpu.make_async_copy(buf1, out_slice(nchunk - 1), wsem1).wait()

    return gather_kernel


def kernel(sen, word_embeddings):
    batch, hist = sen.shape
    n_rows = batch * hist
    rows_per_w = n_rows // NW
    idx = sen.reshape(NW, rows_per_w // G, G)
    one = (sen[0, 0] * 0 + 1).astype(jnp.float32)
    table = jnp.pad(word_embeddings * one, ((0, 0), (0, DP - D)))
    out = _make_gather(n_rows)(idx, table)
    return (out[:, :D] * one).reshape(batch, hist, D, 1)


# speedup vs baseline: 1.1285x; 1.1285x over previous
import functools

import jax
import jax.numpy as jnp
from jax import lax
from jax.experimental import pallas as pl
from jax.experimental.pallas import tpu as pltpu
from jax.experimental.pallas import tpu_sc as plsc

D = 100
DP = 128
NC = 2
NS = 16
NW = NC * NS
G = 128
K = 2
CHUNK = K * G
VB = 512


def _eye(n):
    return (jax.lax.broadcasted_iota(jnp.int32, (n, n), 0)
            == jax.lax.broadcasted_iota(jnp.int32, (n, n), 1)
            ).astype(jnp.float32)


def _table_prep(wt, vp):
    def body(wt_ref, out_ref):
        ey = _eye(VB)
        blk = jnp.concatenate(
            [wt_ref[...], jnp.zeros((DP - D, VB), jnp.float32)], axis=0)
        out_ref[...] = jax.lax.dot_general(
            ey, blk, (((1,), (1,)), ((), ())),
            preferred_element_type=jnp.float32)

    return pl.pallas_call(
        body,
        grid=(vp // VB,),
        in_specs=[pl.BlockSpec((D, VB), lambda i: (0, i))],
        out_specs=pl.BlockSpec((VB, DP), lambda i: (i, 0)),
        out_shape=jax.ShapeDtypeStruct((vp, DP), jnp.float32),
    )(wt)


def _out_prep(x, hist, batch):
    def body(x_ref, o_ref):
        ey = _eye(DP)
        c = jax.lax.dot_general(
            ey, x_ref[0], (((1,), (1,)), ((), ())),
            preferred_element_type=jnp.float32)
        o_ref[...] = c.reshape(DP, batch // 128, 128)[:D].reshape(
            D * batch // 128, 128)

    rows = D * batch // 128
    return pl.pallas_call(
        body,
        grid=(hist,),
        in_specs=[pl.BlockSpec((1, batch, DP), lambda l: (l, 0, 0))],
        out_specs=pl.BlockSpec((rows, 128), lambda l: (l, 0)),
        out_shape=jax.ShapeDtypeStruct((hist * rows, 128), jnp.float32),
    )(x)


def _make_gather(n_rows):
    rows_per_w = n_rows // NW
    ng = rows_per_w // G
    nchunk = ng // K
    assert n_rows % (NW * G) == 0 and ng % (2 * K) == 0

    mesh = plsc.VectorSubcoreMesh(core_axis_name="c", subcore_axis_name="s")

    @functools.partial(
        pl.kernel,
        out_type=jax.ShapeDtypeStruct((n_rows, DP), jnp.float32),
        mesh=mesh,
        scratch_types=[
            pltpu.VMEM((ng, G), jnp.int32),
            pltpu.VMEM((CHUNK, DP), jnp.float32),
            pltpu.VMEM((CHUNK, DP), jnp.float32),
            pltpu.SemaphoreType.DMA,
            pltpu.SemaphoreType.DMA,
            pltpu.SemaphoreType.DMA,
            pltpu.SemaphoreType.DMA,
        ],
    )
    def gather_kernel(idx_hbm, table_hbm, out_hbm,
                      idx_v, buf0, buf1, gsem0, gsem1, wsem0, wsem1):
        wid = lax.axis_index("s") * NC + lax.axis_index("c")
        row0 = wid * rows_per_w

        pltpu.sync_copy(idx_hbm.at[wid], idx_v)

        def out_slice(c):
            return out_hbm.at[pl.ds(row0 + c * CHUNK, CHUNK)]

        def fire_gathers(c, buf, sem):
            return [
                pltpu.async_copy(
                    table_hbm.at[idx_v.at[c * K + j]],
                    buf.at[pl.ds(j * G, G)],
                    sem,
                )
                for j in range(K)
            ]

        def body(t, _):
            a = 2 * t

            @pl.when(t > 0)
            def _drain_prev():
                pltpu.make_async_copy(buf0, out_slice(a - 2), wsem0).wait()
                pltpu.make_async_copy(buf1, out_slice(a - 1), wsem1).wait()

            ha = fire_gathers(a, buf0, gsem0)
            hb = fire_gathers(a + 1, buf1, gsem1)
            for h in ha:
                h.wait()
            pltpu.async_copy(buf0, out_slice(a), wsem0)
            for h in hb:
                h.wait()
            pltpu.async_copy(buf1, out_slice(a + 1), wsem1)
            return 0

        lax.fori_loop(0, nchunk // 2, body, 0)
        pltpu.make_async_copy(buf0, out_slice(nchunk - 2), wsem0).wait()
        pltpu.make_async_copy(buf1, out_slice(nchunk - 1), wsem1).wait()

    return gather_kernel


def kernel(sen, word_embeddings):
    batch, hist = sen.shape
    vocab = word_embeddings.shape[0]
    vp = -(-vocab // VB) * VB
    n_rows = batch * hist
    rows_per_w = n_rows // NW

    idx = jnp.transpose(sen).reshape(NW, rows_per_w // G, G)
    table = _table_prep(jnp.transpose(word_embeddings), vp)
    out = _make_gather(n_rows)(idx, table)
    flat = _out_prep(out.reshape(hist, batch, DP), hist, batch)
    y = flat.reshape(hist, D, batch // 128, 128)
    y = jnp.transpose(y, (2, 3, 0, 1))
    return y.reshape(batch, hist, D, 1)
